# baseline (speedup 1.0000x reference)
import functools

import jax
import jax.numpy as jnp
from jax import lax
from jax.experimental import pallas as pl
from jax.experimental.pallas import tpu as pltpu
from jax.experimental.pallas import tpu_sc as plsc

_B, _N, _D = 1024, 200, 64
_NC, _NS = 2, 16
_NW = _NC * _NS
_BPW = _B // _NW
_C0, _C1 = 128, 72
_LOOK = 2
_NB = 4
_NCH = 13

_MESH = plsc.VectorSubcoreMesh(core_axis_name="c", subcore_axis_name="s")
_PARAMS = pltpu.CompilerParams(use_tc_tiling_on_sc=False,
                               needs_layout_passes=False)


def _worker_base():
  wid = lax.axis_index("s") * _NC + lax.axis_index("c")
  return wid * _BPW


@functools.partial(
    pl.kernel, mesh=_MESH, compiler_params=_PARAMS,
    scratch_types=(
        [pltpu.VMEM((_N, _D), jnp.float32) for _ in range(_NB)]
        + [pltpu.SemaphoreType.DMA for _ in range(2 * _NB)]
    ),
)
def _sc_dis(dis_hbm, out_hbm, *scratch):
  disb = scratch[:_NB]
  gsems = scratch[_NB:2 * _NB]
  wsems = scratch[2 * _NB:3 * _NB]
  base = _worker_base()

  gath = [None] * _NB
  wr = [None] * _NB
  for j in range(_LOOK):
    gath[j % _NB] = pltpu.async_copy(dis_hbm.at[base + j], disb[j % _NB],
                                     gsems[j % _NB])
  for i in range(_BPW):
    j = i + _LOOK
    if j < _BPW:
      slot = j % _NB
      if wr[slot] is not None:
        wr[slot].wait()
      gath[slot] = pltpu.async_copy(dis_hbm.at[base + j], disb[slot],
                                    gsems[slot])
    slot = i % _NB
    gath[slot].wait()
    wr[slot] = pltpu.async_copy(disb[slot],
                                out_hbm.at[base + i, :, pl.ds(_D, _D)],
                                wsems[slot])
  for w in wr:
    if w is not None:
      w.wait()


@functools.partial(
    pl.kernel, mesh=_MESH, compiler_params=_PARAMS,
    scratch_types=(
        [pltpu.VMEM((_BPW, _N), jnp.float32)]
        + [pltpu.VMEM((_N,), jnp.int32) for _ in range(_NB)]
        + [pltpu.VMEM((_N, _D), jnp.float32) for _ in range(_NB)]
        + [pltpu.SemaphoreType.DMA for _ in range(2 * _NB)]
    ),
)
def _sc_men(men_hbm, idx_hbm, out_hbm, idx_v, *scratch):
  lists = scratch[:_NB]
  rows = scratch[_NB:2 * _NB]
  gsems = scratch[2 * _NB:3 * _NB]
  wsems = scratch[3 * _NB:4 * _NB]
  base = _worker_base()
  pltpu.sync_copy(idx_hbm.at[pl.ds(base, _BPW)], idx_v)

  def build_list(j):
    slot = j % _NB
    for k in range(_NCH):
      r0 = min(16 * k, _N - 16)
      lists[slot][pl.ds(r0, 16)] = plsc.bitcast(idx_v[j, pl.ds(r0, 16)],
                                                jnp.int32)

  def fire(j):
    slot = j % _NB
    b = base + j
    c0 = pltpu.async_copy(men_hbm.at[b].at[lists[slot].at[pl.ds(0, _C0)]],
                          rows[slot].at[pl.ds(0, _C0)], gsems[slot])
    c1 = pltpu.async_copy(men_hbm.at[b].at[lists[slot].at[pl.ds(_C0, _C1)]],
                          rows[slot].at[pl.ds(_C0, _C1)], gsems[slot])
    return (c0, c1)

  gath = [None] * _NB
  wr = [None] * _NB
  for j in range(_LOOK):
    build_list(j)
    gath[j % _NB] = fire(j)
  for i in range(_BPW):
    j = i + _LOOK
    if j < _BPW:
      slot = j % _NB
      if wr[slot] is not None:
        wr[slot].wait()
      build_list(j)
      gath[slot] = fire(j)
    slot = i % _NB
    for c in gath[slot]:
      c.wait()
    wr[slot] = pltpu.async_copy(rows[slot],
                                out_hbm.at[base + i, :, pl.ds(0, _D)],
                                wsems[slot])
  for w in wr:
    if w is not None:
      w.wait()


def kernel(dis_entity, men_state, dis_entity_mark):
  idx_f = lax.bitcast_convert_type(dis_entity_mark.astype(jnp.int32),
                                   jnp.float32)
  out_ref = jax.empty_ref(
      jax.ShapeDtypeStruct((_B, _N, 2 * _D), jnp.float32))
  out_ref[:, :, _D:] = dis_entity
  _sc_men(men_state, idx_f, out_ref)
  return out_ref[...]


# speedup vs baseline: 2.9304x; 2.9304x over previous
import functools

import jax
import jax.numpy as jnp
from jax import lax
from jax.experimental import pallas as pl
from jax.experimental.pallas import tpu as pltpu
from jax.experimental.pallas import tpu_sc as plsc

_B, _N, _D = 1024, 200, 64
_NC, _NS = 2, 16
_NW = _NC * _NS
_BPW = _B // _NW
_C0, _C1 = 128, 72
_LOOK = 2
_NB = 4
_NCH = 13

_MESH = plsc.VectorSubcoreMesh(core_axis_name="c", subcore_axis_name="s")
_PARAMS = pltpu.CompilerParams(use_tc_tiling_on_sc=False,
                               needs_layout_passes=False)


def _worker_base():
  wid = lax.axis_index("s") * _NC + lax.axis_index("c")
  return wid * _BPW


@functools.partial(
    pl.kernel, mesh=_MESH, compiler_params=_PARAMS,
    scratch_types=(
        [pltpu.VMEM((_N, _D), jnp.float32) for _ in range(_NB)]
        + [pltpu.SemaphoreType.DMA for _ in range(2 * _NB)]
    ),
)
def _sc_dis(dis_hbm, out_hbm, *scratch):
  disb = scratch[:_NB]
  gsems = scratch[_NB:2 * _NB]
  wsems = scratch[2 * _NB:3 * _NB]
  base = _worker_base()

  gath = [None] * _NB
  wr = [None] * _NB
  for j in range(_LOOK):
    gath[j % _NB] = pltpu.async_copy(dis_hbm.at[base + j], disb[j % _NB],
                                     gsems[j % _NB])
  for i in range(_BPW):
    j = i + _LOOK
    if j < _BPW:
      slot = j % _NB
      if wr[slot] is not None:
        wr[slot].wait()
      gath[slot] = pltpu.async_copy(dis_hbm.at[base + j], disb[slot],
                                    gsems[slot])
    slot = i % _NB
    gath[slot].wait()
    wr[slot] = pltpu.async_copy(disb[slot],
                                out_hbm.at[base + i, :, pl.ds(_D, _D)],
                                wsems[slot])
  for w in wr:
    if w is not None:
      w.wait()


@functools.partial(
    pl.kernel, mesh=_MESH, compiler_params=_PARAMS,
    scratch_types=(
        [pltpu.VMEM((_BPW, _N), jnp.float32)]
        + [pltpu.VMEM((_N,), jnp.int32) for _ in range(_NB)]
        + [pltpu.VMEM((_N, _D), jnp.float32) for _ in range(_NB)]
        + [pltpu.SemaphoreType.DMA for _ in range(2 * _NB)]
    ),
)
def _sc_men(men_hbm, idx_hbm, out_hbm, idx_v, *scratch):
  lists = scratch[:_NB]
  rows = scratch[_NB:2 * _NB]
  gsems = scratch[2 * _NB:3 * _NB]
  wsems = scratch[3 * _NB:4 * _NB]
  base = _worker_base()
  pltpu.sync_copy(idx_hbm.at[pl.ds(base, _BPW)], idx_v)

  def build_list(j):
    slot = j % _NB
    for k in range(_NCH):
      r0 = min(16 * k, _N - 16)
      lists[slot][pl.ds(r0, 16)] = plsc.bitcast(idx_v[j, pl.ds(r0, 16)],
                                                jnp.int32)

  def fire(j):
    slot = j % _NB
    b = base + j
    c0 = pltpu.async_copy(men_hbm.at[b].at[lists[slot].at[pl.ds(0, _C0)]],
                          rows[slot].at[pl.ds(0, _C0)], gsems[slot])
    c1 = pltpu.async_copy(men_hbm.at[b].at[lists[slot].at[pl.ds(_C0, _C1)]],
                          rows[slot].at[pl.ds(_C0, _C1)], gsems[slot])
    return (c0, c1)

  gath = [None] * _NB
  wr = [None] * _NB
  for j in range(_LOOK):
    build_list(j)
    gath[j % _NB] = fire(j)
  for i in range(_BPW):
    j = i + _LOOK
    if j < _BPW:
      slot = j % _NB
      if wr[slot] is not None:
        wr[slot].wait()
      build_list(j)
      gath[slot] = fire(j)
    slot = i % _NB
    for c in gath[slot]:
      c.wait()
    wr[slot] = pltpu.async_copy(rows[slot],
                                out_hbm.at[base + i, :, pl.ds(0, _D)],
                                wsems[slot])
  for w in wr:
    if w is not None:
      w.wait()


def kernel(dis_entity, men_state, dis_entity_mark):
  idx_f = lax.bitcast_convert_type(dis_entity_mark.astype(jnp.int32),
                                   jnp.float32)
  out_ref = jax.empty_ref(
      jax.ShapeDtypeStruct((_B, _N, 2 * _D), jnp.float32))
  _sc_dis(dis_entity, out_ref)
  _sc_men(men_state, idx_f, out_ref)
  return out_ref[...]


# speedup vs baseline: 2.9423x; 1.0041x over previous
import functools

import jax
import jax.numpy as jnp
from jax import lax
from jax.experimental import pallas as pl
from jax.experimental.pallas import tpu as pltpu
from jax.experimental.pallas import tpu_sc as plsc

_B, _N, _D = 1024, 200, 64
_NC, _NS = 2, 16
_NW = _NC * _NS
_BPW = _B // _NW
_C0, _C1 = 128, 72
_LOOK = 3
_NB = 6
_NCH = 13

_MESH = plsc.VectorSubcoreMesh(core_axis_name="c", subcore_axis_name="s")
_PARAMS = pltpu.CompilerParams(use_tc_tiling_on_sc=False,
                               needs_layout_passes=False)


def _worker_base():
  wid = lax.axis_index("s") * _NC + lax.axis_index("c")
  return wid * _BPW


@functools.partial(
    pl.kernel, mesh=_MESH, compiler_params=_PARAMS,
    scratch_types=(
        [pltpu.VMEM((_N, _D), jnp.float32) for _ in range(_NB)]
        + [pltpu.SemaphoreType.DMA for _ in range(2 * _NB)]
    ),
)
def _sc_dis(dis_hbm, out_hbm, *scratch):
  disb = scratch[:_NB]
  gsems = scratch[_NB:2 * _NB]
  wsems = scratch[2 * _NB:3 * _NB]
  base = _worker_base()

  gath = [None] * _NB
  wr = [None] * _NB
  for j in range(_LOOK):
    gath[j % _NB] = pltpu.async_copy(dis_hbm.at[base + j], disb[j % _NB],
                                     gsems[j % _NB])
  for i in range(_BPW):
    j = i + _LOOK
    if j < _BPW:
      slot = j % _NB
      if wr[slot] is not None:
        wr[slot].wait()
      gath[slot] = pltpu.async_copy(dis_hbm.at[base + j], disb[slot],
                                    gsems[slot])
    slot = i % _NB
    gath[slot].wait()
    wr[slot] = pltpu.async_copy(disb[slot],
                                out_hbm.at[base + i, :, pl.ds(_D, _D)],
                                wsems[slot])
  for w in wr:
    if w is not None:
      w.wait()


@functools.partial(
    pl.kernel, mesh=_MESH, compiler_params=_PARAMS,
    scratch_types=(
        [pltpu.VMEM((_BPW, _N), jnp.float32)]
        + [pltpu.VMEM((_N,), jnp.int32) for _ in range(_NB)]
        + [pltpu.VMEM((_N, _D), jnp.float32) for _ in range(_NB)]
        + [pltpu.SemaphoreType.DMA for _ in range(2 * _NB)]
    ),
)
def _sc_men(men_hbm, idx_hbm, out_hbm, idx_v, *scratch):
  lists = scratch[:_NB]
  rows = scratch[_NB:2 * _NB]
  gsems = scratch[2 * _NB:3 * _NB]
  wsems = scratch[3 * _NB:4 * _NB]
  base = _worker_base()
  pltpu.sync_copy(idx_hbm.at[pl.ds(base, _BPW)], idx_v)

  def build_list(j):
    slot = j % _NB
    for k in range(_NCH):
      r0 = min(16 * k, _N - 16)
      lists[slot][pl.ds(r0, 16)] = plsc.bitcast(idx_v[j, pl.ds(r0, 16)],
                                                jnp.int32)

  def fire(j):
    slot = j % _NB
    b = base + j
    c0 = pltpu.async_copy(men_hbm.at[b].at[lists[slot].at[pl.ds(0, _C0)]],
                          rows[slot].at[pl.ds(0, _C0)], gsems[slot])
    c1 = pltpu.async_copy(men_hbm.at[b].at[lists[slot].at[pl.ds(_C0, _C1)]],
                          rows[slot].at[pl.ds(_C0, _C1)], gsems[slot])
    return (c0, c1)

  gath = [None] * _NB
  wr = [None] * _NB
  for j in range(_LOOK):
    build_list(j)
    gath[j % _NB] = fire(j)
  for i in range(_BPW):
    j = i + _LOOK
    if j < _BPW:
      slot = j % _NB
      if wr[slot] is not None:
        wr[slot].wait()
      build_list(j)
      gath[slot] = fire(j)
    slot = i % _NB
    for c in gath[slot]:
      c.wait()
    wr[slot] = pltpu.async_copy(rows[slot],
                                out_hbm.at[base + i, :, pl.ds(0, _D)],
                                wsems[slot])
  for w in wr:
    if w is not None:
      w.wait()


def kernel(dis_entity, men_state, dis_entity_mark):
  idx_f = lax.bitcast_convert_type(dis_entity_mark.astype(jnp.int32),
                                   jnp.float32)
  out_ref = jax.empty_ref(
      jax.ShapeDtypeStruct((_B, _N, 2 * _D), jnp.float32))
  _sc_dis(dis_entity, out_ref)
  _sc_men(men_state, idx_f, out_ref)
  return out_ref[...]


# speedup vs baseline: 2.9485x; 1.0021x over previous
import functools

import jax
import jax.numpy as jnp
from jax import lax
from jax.experimental import pallas as pl
from jax.experimental.pallas import tpu as pltpu
from jax.experimental.pallas import tpu_sc as plsc

_B, _N, _D = 1024, 200, 64
_NC, _NS = 2, 16
_NW = _NC * _NS
_BPW = _B // _NW
_C0, _C1 = 128, 72
_LOOK = 4
_NB = 8
_NCH = 13

_MESH = plsc.VectorSubcoreMesh(core_axis_name="c", subcore_axis_name="s")
_PARAMS = pltpu.CompilerParams(use_tc_tiling_on_sc=False,
                               needs_layout_passes=False)


def _worker_base():
  wid = lax.axis_index("s") * _NC + lax.axis_index("c")
  return wid * _BPW


@functools.partial(
    pl.kernel, mesh=_MESH, compiler_params=_PARAMS,
    scratch_types=(
        [pltpu.VMEM((_N, _D), jnp.float32) for _ in range(_NB)]
        + [pltpu.SemaphoreType.DMA for _ in range(2 * _NB)]
    ),
)
def _sc_dis(dis_hbm, out_hbm, *scratch):
  disb = scratch[:_NB]
  gsems = scratch[_NB:2 * _NB]
  wsems = scratch[2 * _NB:3 * _NB]
  base = _worker_base()

  gath = [None] * _NB
  wr = [None] * _NB
  for j in range(_LOOK):
    gath[j % _NB] = pltpu.async_copy(dis_hbm.at[base + j], disb[j % _NB],
                                     gsems[j % _NB])
  for i in range(_BPW):
    j = i + _LOOK
    if j < _BPW:
      slot = j % _NB
      if wr[slot] is not None:
        wr[slot].wait()
      gath[slot] = pltpu.async_copy(dis_hbm.at[base + j], disb[slot],
                                    gsems[slot])
    slot = i % _NB
    gath[slot].wait()
    wr[slot] = pltpu.async_copy(disb[slot],
                                out_hbm.at[base + i, :, pl.ds(_D, _D)],
                                wsems[slot])
  for w in wr:
    if w is not None:
      w.wait()


@functools.partial(
    pl.kernel, mesh=_MESH, compiler_params=_PARAMS,
    scratch_types=(
        [pltpu.VMEM((_BPW, _N), jnp.float32)]
        + [pltpu.VMEM((_N,), jnp.int32) for _ in range(_NB)]
        + [pltpu.VMEM((_N, _D), jnp.float32) for _ in range(_NB)]
        + [pltpu.SemaphoreType.DMA for _ in range(2 * _NB)]
    ),
)
def _sc_men(men_hbm, idx_hbm, out_hbm, idx_v, *scratch):
  lists = scratch[:_NB]
  rows = scratch[_NB:2 * _NB]
  gsems = scratch[2 * _NB:3 * _NB]
  wsems = scratch[3 * _NB:4 * _NB]
  base = _worker_base()
  pltpu.sync_copy(idx_hbm.at[pl.ds(base, _BPW)], idx_v)

  def build_list(j):
    slot = j % _NB
    for k in range(_NCH):
      r0 = min(16 * k, _N - 16)
      lists[slot][pl.ds(r0, 16)] = plsc.bitcast(idx_v[j, pl.ds(r0, 16)],
                                                jnp.int32)

  def fire(j):
    slot = j % _NB
    b = base + j
    c0 = pltpu.async_copy(men_hbm.at[b].at[lists[slot].at[pl.ds(0, _C0)]],
                          rows[slot].at[pl.ds(0, _C0)], gsems[slot])
    c1 = pltpu.async_copy(men_hbm.at[b].at[lists[slot].at[pl.ds(_C0, _C1)]],
                          rows[slot].at[pl.ds(_C0, _C1)], gsems[slot])
    return (c0, c1)

  gath = [None] * _NB
  wr = [None] * _NB
  for j in range(_LOOK):
    build_list(j)
    gath[j % _NB] = fire(j)
  for i in range(_BPW):
    j = i + _LOOK
    if j < _BPW:
      slot = j % _NB
      if wr[slot] is not None:
        wr[slot].wait()
      build_list(j)
      gath[slot] = fire(j)
    slot = i % _NB
    for c in gath[slot]:
      c.wait()
    wr[slot] = pltpu.async_copy(rows[slot],
                                out_hbm.at[base + i, :, pl.ds(0, _D)],
                                wsems[slot])
  for w in wr:
    if w is not None:
      w.wait()


def kernel(dis_entity, men_state, dis_entity_mark):
  idx_f = lax.bitcast_convert_type(dis_entity_mark.astype(jnp.int32),
                                   jnp.float32)
  out_ref = jax.empty_ref(
      jax.ShapeDtypeStruct((_B, _N, 2 * _D), jnp.float32))
  _sc_dis(dis_entity, out_ref)
  _sc_men(men_state, idx_f, out_ref)
  return out_ref[...]
